# baseline (speedup 1.0000x reference)
import functools

import jax
import jax.numpy as jnp
from jax import lax
from jax.experimental import pallas as pl
from jax.experimental.pallas import tpu as pltpu
from jax.experimental.pallas import tpu_sc as plsc

_IPG = 128
_K = 8
_CHUNK = _K * _IPG
_PD = 128


@functools.lru_cache(maxsize=None)
def _make_gather(total: int, dim: int, nrows: int):
    info = plsc.get_sparse_core_info()
    nc, ns = info.num_cores, info.num_subcores
    nw = nc * ns
    assert total % (nw * _CHUNK) == 0
    nb = total // (nw * _CHUNK)
    assert nb % 2 == 1 and nb >= 3
    mesh = plsc.VectorSubcoreMesh(core_axis_name="c", subcore_axis_name="s")

    @functools.partial(
        pl.kernel,
        mesh=mesh,
        out_type=jax.ShapeDtypeStruct((nw, nb, _CHUNK, _PD), jnp.float32),
        scratch_types=[
            pltpu.VMEM((nb * _K, _IPG), jnp.int32),
            pltpu.VMEM((_CHUNK, dim), jnp.float32),
            pltpu.VMEM((_CHUNK, dim), jnp.float32),
            pltpu.SemaphoreType.DMA,
            pltpu.SemaphoreType.DMA,
            pltpu.SemaphoreType.DMA,
            pltpu.SemaphoreType.DMA,
        ],
        compiler_params=pltpu.CompilerParams(use_tc_tiling_on_sc=False),
    )
    def gather(idx_hbm, t128_hbm, out_hbm, idx_v, rows0, rows1, g0, g1,
               w0, w1):
        wid = lax.axis_index("s") * nc + lax.axis_index("c")
        rows = (rows0, rows1)
        g_sem = (g0, g1)
        w_sem = (w0, w1)
        tlin = t128_hbm

        def fire(cur, buf, sem):
            for j in range(_K):
                pltpu.async_copy(
                    tlin.at[idx_v.at[cur * _K + j]],
                    buf.at[pl.ds(j * _IPG, _IPG)],
                    sem,
                )

        def out_slab(b):
            return out_hbm.at[wid, b].at[:, pl.ds(0, dim)]

        def drain_gather(p):
            pltpu.make_async_copy(out_slab(0), rows[p], g_sem[p]).wait()

        def drain_wb(p):
            pltpu.make_async_copy(rows[p], out_slab(0), w_sem[p]).wait()

        pltpu.sync_copy(idx_hbm.at[wid], idx_v)

        fire(0, rows[0], g_sem[0])

        def body(g, carry):
            for b in (0, 1):
                cur = 1 + 2 * g + b
                cb = 1 - b
                pb = b

                @pl.when(cur >= 2)
                def _():
                    drain_wb(cb)
                fire(cur, rows[cb], g_sem[cb])
                drain_gather(pb)
                pltpu.async_copy(rows[pb], out_slab(cur - 1), w_sem[pb])
            return carry

        lax.fori_loop(0, (nb - 1) // 2, body, 0)

        drain_gather(0)
        pltpu.async_copy(rows[0], out_slab(nb - 1), w_sem[0])
        drain_wb(1)
        drain_wb(0)

    return gather


def kernel(x, table):
    b, h = x.shape
    n, d = table.shape
    total = b * h
    gather = _make_gather(total, d, n)
    info = plsc.get_sparse_core_info()
    nw = info.num_cores * info.num_subcores
    idx = x.astype(jnp.int32).reshape(nw, total // (nw * _IPG), _IPG)
    t128 = jax.lax.optimization_barrier(table.reshape(n * d // _PD, _PD))
    tlin = t128.reshape(n, d)
    out = gather(idx, tlin)
    return out.reshape(b, h, _PD)[:, :, :d]


# speedup vs baseline: 1.0182x; 1.0182x over previous
import functools

import jax
import jax.numpy as jnp
from jax import lax
from jax.experimental import pallas as pl
from jax.experimental.pallas import tpu as pltpu
from jax.experimental.pallas import tpu_sc as plsc

_IPG = 128
_K = 8
_CHUNK = _K * _IPG
_PD = 128


@functools.lru_cache(maxsize=None)
def _make_gather(total: int, dim: int):
    info = plsc.get_sparse_core_info()
    nc, ns = info.num_cores, info.num_subcores
    nw = nc * ns
    assert total % (nw * _CHUNK) == 0
    nb = total // (nw * _CHUNK)
    assert nb % 2 == 1 and nb >= 3
    mesh = plsc.VectorSubcoreMesh(core_axis_name="c", subcore_axis_name="s")

    @functools.partial(
        pl.kernel,
        mesh=mesh,
        out_type=jax.ShapeDtypeStruct((nw, nb, _CHUNK, _PD), jnp.float32),
        scratch_types=[
            pltpu.VMEM((nb * _K, _IPG), jnp.int32),
            pltpu.VMEM((_CHUNK, dim), jnp.float32),
            pltpu.VMEM((_CHUNK, dim), jnp.float32),
            pltpu.SemaphoreType.DMA,
            pltpu.SemaphoreType.DMA,
            pltpu.SemaphoreType.DMA,
            pltpu.SemaphoreType.DMA,
        ],
        compiler_params=pltpu.CompilerParams(use_tc_tiling_on_sc=False),
    )
    def gather(idx_hbm, t4_hbm, out_hbm, idx_v, rows0, rows1, g0, g1,
               w0, w1):
        wid = lax.axis_index("s") * nc + lax.axis_index("c")
        rows = (rows0, rows1)
        g_sem = (g0, g1)
        w_sem = (w0, w1)
        tlin = t4_hbm

        def fire(cur, buf, sem):
            for j in range(_K):
                pltpu.async_copy(
                    tlin.at[idx_v.at[cur * _K + j]],
                    buf.at[pl.ds(j * _IPG, _IPG)],
                    sem,
                )

        def out_slab(b):
            return out_hbm.at[wid, b].at[:, pl.ds(0, dim)]

        def drain_gather(p):
            pltpu.make_async_copy(out_slab(0), rows[p], g_sem[p]).wait()

        def drain_wb(p):
            pltpu.make_async_copy(rows[p], out_slab(0), w_sem[p]).wait()

        pltpu.sync_copy(idx_hbm.at[wid], idx_v)

        fire(0, rows[0], g_sem[0])

        def body(g, carry):
            for b in (0, 1):
                cur = 1 + 2 * g + b
                cb = 1 - b
                pb = b

                @pl.when(cur >= 2)
                def _():
                    drain_wb(cb)
                fire(cur, rows[cb], g_sem[cb])
                drain_gather(pb)
                pltpu.async_copy(rows[pb], out_slab(cur - 1), w_sem[pb])
            return carry

        lax.fori_loop(0, (nb - 1) // 2, body, 0)

        drain_gather(0)
        pltpu.async_copy(rows[0], out_slab(nb - 1), w_sem[0])
        drain_wb(1)
        drain_wb(0)

    return gather


def kernel(x, table):
    b, h = x.shape
    n, d = table.shape
    total = b * h
    gather = _make_gather(total, d)
    info = plsc.get_sparse_core_info()
    nw = info.num_cores * info.num_subcores
    ratio = _PD // d
    idx = (x.astype(jnp.int32) * ratio).reshape(nw, total // (nw * _IPG),
                                                _IPG)
    t4 = jnp.pad(table, ((0, 0), (0, _PD - d))).reshape(n * ratio, d)
    out = gather(idx, t4)
    return out.reshape(b, h, _PD)[:, :, :d]
